# Initial kernel scaffold; baseline (speedup 1.0000x reference)
import jax
import jax.numpy as jnp
from jax.experimental import pallas as pl


def kernel(input, weight, bias, indx_seqs):
    raise NotImplementedError("write your pallas kernel here")


# speedup vs baseline: 4.5576x; 4.5576x over previous
import functools

import jax
import jax.numpy as jnp
from jax import lax
from jax.experimental import pallas as pl
from jax.experimental.pallas import tpu as pltpu
from jax.experimental.pallas import tpu_sc as plsc

O = 2048
IL = 4096
K = 32
B = 1024

NC = 2
NS = 16
NW = NC * NS
RPT = O // NW
C = 4
NCH = RPT // C


def _sc_body(w_hbm, idx_hbm, mt_hbm, idx_v, w_v, buf0, buf1, sem0, sem1):
    wid = lax.axis_index("c") * NS + lax.axis_index("s")
    rbase = wid * RPT
    ebase = rbase * K
    pltpu.sync_copy(idx_hbm.at[pl.ds(ebase, RPT * K)], idx_v)
    pltpu.sync_copy(w_hbm.at[pl.ds(ebase, RPT * K)], w_v)

    z16 = jnp.zeros((16,), jnp.float32)
    bufs = (buf0, buf1)
    sems = (sem0, sem1)

    def _memset(t, carry):
        buf0[pl.ds(t * 16, 16)] = z16
        buf1[pl.ds(t * 16, 16)] = z16
        return carry
    lax.fori_loop(0, C * IL // 16, _memset, 0)

    def _for_chunk(c, fn):
        b = bufs[c % 2]
        for j in range(C):
            r = (c * C + j) * K
            for h in range(2):
                o = r + h * 16
                iv = idx_v[pl.ds(o, 16)] + (j * IL)
                fn(b, iv, w_v[pl.ds(o, 16)])

    handles = [None, None]
    for c in range(NCH):
        s = c % 2
        if handles[s] is not None:
            handles[s].wait()
            _for_chunk(c - 2, lambda b, iv, wv: plsc.store_scatter(b, [iv], z16))
        _for_chunk(c, lambda b, iv, wv: plsc.addupdate_scatter(b, [iv], wv))
        dst = mt_hbm.at[pl.ds((rbase + c * C) * IL, C * IL)]
        handles[s] = pltpu.async_copy(bufs[s], dst, sems[s])
    handles[0].wait()
    handles[1].wait()


_build_mt = pl.kernel(
    _sc_body,
    out_type=jax.ShapeDtypeStruct((O * IL,), jnp.float32),
    mesh=plsc.VectorSubcoreMesh(
        core_axis_name="c", subcore_axis_name="s", num_cores=NC, num_subcores=NS
    ),
    compiler_params=pltpu.CompilerParams(needs_layout_passes=False),
    scratch_types=[
        pltpu.VMEM((RPT * K,), jnp.int32),
        pltpu.VMEM((RPT * K,), jnp.float32),
        pltpu.VMEM((C * IL,), jnp.float32),
        pltpu.VMEM((C * IL,), jnp.float32),
        pltpu.SemaphoreType.DMA,
        pltpu.SemaphoreType.DMA,
    ],
)


BO = 512


def _mm_body(x_ref, mt_ref, b_ref, o_ref):
    acc = lax.dot_general(
        x_ref[...], mt_ref[...],
        (((1,), (1,)), ((), ())),
        preferred_element_type=jnp.float32,
    )
    o_ref[...] = acc + b_ref[...]


def _matmul(x, mt, bias2d):
    return pl.pallas_call(
        _mm_body,
        grid=(O // BO,),
        in_specs=[
            pl.BlockSpec((B, IL), lambda i: (0, 0)),
            pl.BlockSpec((BO, IL), lambda i: (i, 0)),
            pl.BlockSpec((1, BO), lambda i: (0, i)),
        ],
        out_specs=pl.BlockSpec((B, BO), lambda i: (0, i)),
        out_shape=jax.ShapeDtypeStruct((B, O), jnp.float32),
    )(x, mt, bias2d)


def kernel(input, weight, bias, indx_seqs):
    w_flat = weight.reshape(O * K)
    idx_flat = indx_seqs.reshape(O * K)
    mt = _build_mt(w_flat, idx_flat).reshape(O, IL)
    return _matmul(input, mt, bias.reshape(1, O))


# speedup vs baseline: 7.1798x; 1.5753x over previous
import functools

import jax
import jax.numpy as jnp
from jax import lax
from jax.experimental import pallas as pl
from jax.experimental.pallas import tpu as pltpu
from jax.experimental.pallas import tpu_sc as plsc

O = 2048
IL = 4096
K = 32
B = 1024

NC = 2
NS = 16
NW = NC * NS
RPT = O // NW
C = 4
NCH = RPT // C


def _sc_body(w_hbm, idx_hbm, mt_hbm, idx_v, w_v, buf0, buf1, sem0, sem1):
    wid = lax.axis_index("c") * NS + lax.axis_index("s")
    rbase = wid * RPT
    ebase = rbase * K
    pltpu.sync_copy(idx_hbm.at[pl.ds(ebase, RPT * K)], idx_v)
    pltpu.sync_copy(w_hbm.at[pl.ds(ebase, RPT * K)], w_v)

    z16 = jnp.zeros((16,), jnp.float32)
    bufs = (buf0, buf1)
    sems = (sem0, sem1)

    def _memset(t, carry):
        for j in range(C):
            buf0[j, pl.ds(t * 16, 16)] = z16
            buf1[j, pl.ds(t * 16, 16)] = z16
        return carry
    lax.fori_loop(0, IL // 16, _memset, 0)

    def _for_chunk(c, fn):
        b = bufs[c % 2]
        for j in range(C):
            r = (c * C + j) * K
            jv = jnp.full((16,), j, jnp.int32)
            for h in range(2):
                o = r + h * 16
                fn(b, [jv, idx_v[pl.ds(o, 16)]], w_v[pl.ds(o, 16)])

    handles = [None, None]
    for c in range(NCH):
        s = c % 2
        if handles[s] is not None:
            handles[s].wait()
            _for_chunk(c - 2, lambda b, ix, wv: plsc.store_scatter(b, ix, z16))
        _for_chunk(c, lambda b, ix, wv: plsc.addupdate_scatter(b, ix, wv))
        dst = mt_hbm.at[pl.ds(rbase + c * C, C)]
        handles[s] = pltpu.async_copy(bufs[s], dst, sems[s])
    handles[0].wait()
    handles[1].wait()


_build_mt = pl.kernel(
    _sc_body,
    out_type=jax.ShapeDtypeStruct((O, IL), jnp.float32),
    mesh=plsc.VectorSubcoreMesh(
        core_axis_name="c", subcore_axis_name="s", num_cores=NC, num_subcores=NS
    ),
    compiler_params=pltpu.CompilerParams(needs_layout_passes=False),
    scratch_types=[
        pltpu.VMEM((RPT * K,), jnp.int32),
        pltpu.VMEM((RPT * K,), jnp.float32),
        pltpu.VMEM((C, IL), jnp.float32),
        pltpu.VMEM((C, IL), jnp.float32),
        pltpu.SemaphoreType.DMA,
        pltpu.SemaphoreType.DMA,
    ],
)


BO = 512


def _mm_body(x_ref, mt_ref, b_ref, o_ref):
    acc = lax.dot_general(
        x_ref[...], mt_ref[...],
        (((1,), (1,)), ((), ())),
        preferred_element_type=jnp.float32,
    )
    o_ref[...] = acc + b_ref[...]


def _matmul(x, mt, bias2d):
    return pl.pallas_call(
        _mm_body,
        grid=(O // BO,),
        in_specs=[
            pl.BlockSpec((B, IL), lambda i: (0, 0)),
            pl.BlockSpec((BO, IL), lambda i: (i, 0)),
            pl.BlockSpec((1, BO), lambda i: (0, i)),
        ],
        out_specs=pl.BlockSpec((B, BO), lambda i: (0, i)),
        out_shape=jax.ShapeDtypeStruct((B, O), jnp.float32),
    )(x, mt, bias2d)


def kernel(input, weight, bias, indx_seqs):
    w_flat = weight.reshape(O * K)
    idx_flat = indx_seqs.reshape(O * K)
    mt = _build_mt(w_flat, idx_flat)
    return _matmul(input, mt, bias.reshape(1, O))
